# baseline (speedup 1.0000x reference)
import jax
import jax.numpy as jnp
from jax import lax
from jax.experimental import pallas as pl
from jax.experimental.pallas import tpu as pltpu, tpu_sc as plsc

BATCH, SEQ, D = 4, 8192, 1024
NC, NS = 2, 16
NW = NC * NS
SEQ_PER_W = SEQ // NW
CH = 8
NCH = SEQ_PER_W // CH
XDEPTH = 3


def _sc_body(x_hbm, emb_hbm, out_hbm, *refs):
    ebufs = refs[0:2]
    xbufs = refs[2:2 + XDEPTH]
    ses = refs[2 + XDEPTH:4 + XDEPTH]
    sxs = refs[4 + XDEPTH:4 + 2 * XDEPTH]
    sos = refs[4 + 2 * XDEPTH:4 + 3 * XDEPTH]

    cid = lax.axis_index("c")
    sid = lax.axis_index("s")
    wid = sid * NC + cid
    seq0 = wid * SEQ_PER_W

    def row(c):
        return seq0 + c * CH

    def slab_load(c):
        k = c % XDEPTH
        return (
            pltpu.async_copy(x_hbm.at[pl.ds(0, 2), pl.ds(row(c), CH)],
                             xbufs[k].at[pl.ds(0, 2)], sxs[k]),
            pltpu.async_copy(x_hbm.at[pl.ds(2, 2), pl.ds(row(c), CH)],
                             xbufs[k].at[pl.ds(2, 2)], sxs[k]),
        )

    def slab_store(c):
        k = c % XDEPTH
        return (
            pltpu.async_copy(xbufs[k].at[pl.ds(0, 2)],
                             out_hbm.at[pl.ds(0, 2), pl.ds(row(c), CH)],
                             sos[k]),
            pltpu.async_copy(xbufs[k].at[pl.ds(2, 2)],
                             out_hbm.at[pl.ds(2, 2), pl.ds(row(c), CH)],
                             sos[k]),
        )

    eld = {}
    xld = {}
    xst = {}
    for c in range(2):
        eld[c] = pltpu.async_copy(
            emb_hbm.at[pl.ds(row(c), CH)], ebufs[c % 2], ses[c % 2])
    for c in range(XDEPTH - 1):
        xld[c] = slab_load(c)

    for c in range(NCH):
        k = c % XDEPTH
        eld[c].wait()
        for d in xld[c]:
            d.wait()
        cn = c + XDEPTH - 1
        if cn < NCH:
            if c - 1 >= 0:
                for d in xst[c - 1]:
                    d.wait()
            xld[cn] = slab_load(cn)

        xb = xbufs[k]
        emb_v = ebufs[c % 2]

        @plsc.parallel_loop(0, BATCH)
        def _batch(b):
            @plsc.parallel_loop(0, CH)
            def _row(r):
                @plsc.parallel_loop(0, D, step=16, unroll=8)
                def _add(i):
                    plsc.addupdate(xb.at[b, r, pl.ds(i, 16)],
                                   emb_v[r, pl.ds(i, 16)])

        xst[c] = slab_store(c)

        if c + 2 < NCH:
            eld[c + 2] = pltpu.async_copy(
                emb_hbm.at[pl.ds(row(c + 2), CH)], ebufs[c % 2], ses[c % 2])

    for c in range(NCH - XDEPTH, NCH):
        for d in xst[c]:
            d.wait()


def kernel(x, emb):
    mesh = plsc.VectorSubcoreMesh(core_axis_name="c", subcore_axis_name="s")
    return pl.kernel(
        _sc_body,
        out_type=jax.ShapeDtypeStruct((BATCH, SEQ, D), jnp.float32),
        mesh=mesh,
        scratch_types=(
            [pltpu.VMEM((CH, D), jnp.float32)] * 2
            + [pltpu.VMEM((BATCH, CH, D), jnp.float32)] * XDEPTH
            + [pltpu.SemaphoreType.DMA] * (2 + 2 * XDEPTH)
        ),
    )(x, emb)


# speedup vs baseline: 1.0078x; 1.0078x over previous
import jax
import jax.numpy as jnp
from jax import lax
from jax.experimental import pallas as pl
from jax.experimental.pallas import tpu as pltpu, tpu_sc as plsc

BATCH, SEQ, D = 4, 8192, 1024
NC, NS = 2, 16
NW = NC * NS
SEQ_PER_W = SEQ // NW
CH = 8
NCH = SEQ_PER_W // CH
XDEPTH = 3


def _sc_body(x_hbm, emb_hbm, out_hbm, *refs):
    ebufs = refs[0:2]
    xbufs = refs[2:2 + XDEPTH]
    ses = refs[2 + XDEPTH:4 + XDEPTH]
    sxs = refs[4 + XDEPTH:4 + 2 * XDEPTH]
    sos = refs[4 + 2 * XDEPTH:4 + 3 * XDEPTH]

    cid = lax.axis_index("c")
    sid = lax.axis_index("s")
    wid = sid * NC + cid
    seq0 = wid * SEQ_PER_W

    def row(c):
        return seq0 + c * CH

    eld = {}
    xld = {}
    xst = {}
    for c in range(2):
        eld[c] = pltpu.async_copy(
            emb_hbm.at[pl.ds(row(c), CH)], ebufs[c % 2], ses[c % 2])
    for c in range(XDEPTH - 1):
        xld[c] = pltpu.async_copy(
            x_hbm.at[:, pl.ds(row(c), CH)],
            xbufs[c % XDEPTH], sxs[c % XDEPTH])

    for c in range(NCH):
        k = c % XDEPTH
        eld[c].wait()
        xld[c].wait()
        cn = c + XDEPTH - 1
        if cn < NCH:
            if c - 1 >= 0:
                xst[c - 1].wait()
            xld[cn] = pltpu.async_copy(
                x_hbm.at[:, pl.ds(row(cn), CH)],
                xbufs[cn % XDEPTH], sxs[cn % XDEPTH])

        xb = xbufs[k]
        emb_v = ebufs[c % 2]

        @plsc.parallel_loop(0, BATCH)
        def _batch(b):
            @plsc.parallel_loop(0, CH)
            def _row(r):
                @plsc.parallel_loop(0, D, step=16, unroll=8)
                def _add(i):
                    plsc.addupdate(xb.at[b, r, pl.ds(i, 16)],
                                   emb_v[r, pl.ds(i, 16)])

        xst[c] = pltpu.async_copy(
            xb, out_hbm.at[:, pl.ds(row(c), CH)], sos[k])

        if c + 2 < NCH:
            eld[c + 2] = pltpu.async_copy(
                emb_hbm.at[pl.ds(row(c + 2), CH)], ebufs[c % 2], ses[c % 2])

    for c in range(NCH - XDEPTH, NCH):
        xst[c].wait()


def kernel(x, emb):
    mesh = plsc.VectorSubcoreMesh(core_axis_name="c", subcore_axis_name="s")
    return pl.kernel(
        _sc_body,
        out_type=jax.ShapeDtypeStruct((BATCH, SEQ, D), jnp.float32),
        mesh=mesh,
        scratch_types=(
            [pltpu.VMEM((CH, D), jnp.float32)] * 2
            + [pltpu.VMEM((BATCH, CH, D), jnp.float32)] * XDEPTH
            + [pltpu.SemaphoreType.DMA] * (2 + 2 * XDEPTH)
        ),
    )(x, emb)
